# baseline (speedup 1.0000x reference)
import functools
import math

import jax
import jax.numpy as jnp
from jax import lax
from jax.experimental import pallas as pl
from jax.experimental.pallas import tpu as pltpu
from jax.experimental.pallas import tpu_sc as plsc

N_NODES = 10000
N_EDGES = 320000
HEADS = 8
NCLS = 10

SC_CORES = 2
SC_TILES = 16
NW = SC_CORES * SC_TILES
LANES = 16
CH = 80
NPAD = 10240
RPW = NPAD // NW
ROW_BLOCK = 1000
ROWW = 128
NUMW = 128
DENW = 16
VW = NUMW + DENW
EPAD = 128


def _proj_body(h_ref, w_ref, b_ref, o_ref):
    o_ref[0] = (
        jnp.dot(h_ref[...], w_ref[0], preferred_element_type=jnp.float32)
        + b_ref[0]
    )


def _project(h, wg, bg):
    g_cnt, fin, roww = wg.shape
    bg = bg.reshape(g_cnt, 1, roww)
    grid = (N_NODES // ROW_BLOCK, g_cnt)
    return pl.pallas_call(
        _proj_body,
        grid=grid,
        in_specs=[
            pl.BlockSpec((ROW_BLOCK, fin), lambda i, g: (i, 0)),
            pl.BlockSpec((1, fin, roww), lambda i, g: (g, 0, 0)),
            pl.BlockSpec((1, 1, roww), lambda i, g: (g, 0, 0)),
        ],
        out_specs=pl.BlockSpec((1, ROW_BLOCK, roww), lambda i, g: (g, i, 0)),
        out_shape=jax.ShapeDtypeStruct((g_cnt, N_NODES, roww), jnp.float32),
    )(h, wg, bg)


@functools.lru_cache(maxsize=None)
def _edge_pass(C, groups):
    inv = 1.0 / math.sqrt(C)
    ng = len(groups)
    groups_cnt = groups[0][1]
    hg0 = groups[0][0]
    cnt = sum(g[1] for g in groups)
    numw = cnt * C
    assert numw <= NUMW and cnt <= DENW
    mesh = plsc.VectorSubcoreMesh(core_axis_name="c", subcore_axis_name="s",
                                  num_cores=SC_CORES, num_subcores=SC_TILES)

    def body(qkv_hbm, ssrc_hbm, sdst_hbm, starts_hbm, num_hbm, den_hbm, *scr):
        sidx = scr[0:2]
        didx = scr[2:4]
        gq = scr[4:6]
        gk = scr[6:8]
        gv = scr[8:10]
        rq = scr[10:12]
        rk = scr[12:14]
        rv = scr[14:16]
        accn, accd, starts_v = scr[16:19]
        sems = scr[19:21]

        c = lax.axis_index("c")
        s = lax.axis_index("s")
        w = c * SC_TILES + s
        zeros = jnp.zeros((LANES,), jnp.float32)
        lane = lax.iota(jnp.int32, LANES)

        pltpu.sync_copy(starts_hbm, starts_v)

        def start_at(i):
            v = jnp.zeros((), jnp.int32)
            for j in range(3):
                sl = starts_v[pl.ds(j * LANES, LANES)]
                v = v + jnp.sum(jnp.where(lane + j * LANES == i, sl, 0))
            return v

        e0 = start_at(w)
        e1 = start_at(w + 1)
        e0a = (e0 // 8) * 8
        n_ch = (e1 - e0a + CH - 1) // CH
        base_node = w * RPW

        def zacc(i, _):
            def zc(j, _):
                accn[i, pl.ds(j * LANES, LANES)] = zeros
                return 0
            lax.fori_loop(0, NUMW // LANES, zc, 0)
            accd[pl.ds(i * DENW, DENW)] = zeros
            return 0
        lax.fori_loop(0, RPW, zacc, 0)

        def issue(p, i, ghalf):
            eb = e0a + i * CH
            pltpu.sync_copy(ssrc_hbm.at[pl.ds(eb, CH)], sidx[p])
            pltpu.sync_copy(sdst_hbm.at[pl.ds(eb, CH)], didx[p])
            for g4 in range(CH // LANES):
                sl = pl.ds(g4 * LANES, LANES)
                sv = sidx[p][sl]
                dv = didx[p][sl]
                gq[p][sl] = dv + ghalf * N_NODES
                gk[p][sl] = sv + (2 + ghalf) * N_NODES
                gv[p][sl] = sv + (4 + ghalf) * N_NODES
            pltpu.async_copy(qkv_hbm.at[gq[p]], rq[p], sems[p])
            pltpu.async_copy(qkv_hbm.at[gk[p]], rk[p], sems[p])
            pltpu.async_copy(qkv_hbm.at[gv[p]], rv[p], sems[p])

        def issue2(i):
            eb = e0a + i * CH
            pltpu.sync_copy(ssrc_hbm.at[pl.ds(eb, CH)], sidx[0])
            pltpu.sync_copy(sdst_hbm.at[pl.ds(eb, CH)], didx[0])
            for g4 in range(CH // LANES):
                sl = pl.ds(g4 * LANES, LANES)
                sv = sidx[0][sl]
                dv = didx[0][sl]
                for g, (ghalf, _) in enumerate(groups):
                    gq[g][sl] = dv + ghalf * N_NODES
                    gk[g][sl] = sv + (2 + ghalf) * N_NODES
                    gv[g][sl] = sv + (4 + ghalf) * N_NODES
            for g in range(2):
                pltpu.async_copy(qkv_hbm.at[gq[g]], rq[g], sems[g])
                pltpu.async_copy(qkv_hbm.at[gk[g]], rk[g], sems[g])
                pltpu.async_copy(qkv_hbm.at[gv[g]], rv[g], sems[g])

        def wait(p):
            pltpu.make_async_copy(qkv_hbm.at[gq[p]], rq[p], sems[p]).wait()
            pltpu.make_async_copy(qkv_hbm.at[gk[p]], rk[p], sems[p]).wait()
            pltpu.make_async_copy(qkv_hbm.at[gv[p]], rv[p], sems[p]).wait()

        def head_block(e, r, qrows, krows, vrows, col0, den0, dv):
            hc = cnt if ng == 1 else groups_cnt
            if C == 16:
                for th in range(hc):
                    sl = pl.ds(th * 16, LANES)
                    osl = pl.ds(col0 + th * 16, LANES)
                    d = qrows[e, sl] * krows[e, sl]
                    a = jnp.sum(d) * inv
                    exv = jnp.exp(jnp.broadcast_to(a, (LANES,)))
                    accn[r, osl] = accn[r, osl] + vrows[e, sl] * exv
                    dv = dv + jnp.where(lane == den0 + th, exv, 0.0)
            elif C == 32:
                for th in range(hc):
                    s0 = pl.ds(th * 32, LANES)
                    s1 = pl.ds(th * 32 + 16, LANES)
                    o0 = pl.ds(col0 + th * 32, LANES)
                    o1 = pl.ds(col0 + th * 32 + 16, LANES)
                    d = (qrows[e, s0] * krows[e, s0]
                         + qrows[e, s1] * krows[e, s1])
                    a = jnp.sum(d) * inv
                    exv = jnp.exp(jnp.broadcast_to(a, (LANES,)))
                    accn[r, o0] = accn[r, o0] + vrows[e, s0] * exv
                    accn[r, o1] = accn[r, o1] + vrows[e, s1] * exv
                    dv = dv + jnp.where(lane == den0 + th, exv, 0.0)
            else:
                for tp in range(hc // 2):
                    sl = pl.ds(tp * 16, LANES)
                    osl = pl.ds(col0 + tp * 16, LANES)
                    d = qrows[e, sl] * krows[e, sl]
                    a0 = jnp.sum(jnp.where(lane < 8, d, 0.0)) * inv
                    a1 = jnp.sum(jnp.where(lane >= 8, d, 0.0)) * inv
                    av = jnp.where(lane < 8,
                                   jnp.broadcast_to(a0, (LANES,)),
                                   jnp.broadcast_to(a1, (LANES,)))
                    exv = jnp.exp(av)
                    accn[r, osl] = accn[r, osl] + vrows[e, sl] * exv
                    x0 = jnp.sum(jnp.where(lane == 0, exv, 0.0))
                    x1 = jnp.sum(jnp.where(lane == 8, exv, 0.0))
                    dv = (dv + jnp.where(lane == den0 + 2 * tp, x0, 0.0)
                          + jnp.where(lane == den0 + 2 * tp + 1, x1, 0.0))
            return dv

        def compute(p, i):
            eb = e0a + i * CH
            dxp = didx[p]

            def edge(e, _):
                ge = eb + e

                @pl.when(jnp.logical_and(ge >= e0, ge < e1))
                def _():
                    dsl16 = dxp[pl.ds((e // LANES) * LANES, LANES)]
                    dl = jnp.sum(jnp.where(lane == e % LANES, dsl16, 0))
                    r = dl - base_node
                    dv = jnp.zeros((LANES,), jnp.float32)
                    if ng == 1:
                        dv = head_block(e, r, rq[p], rk[p], rv[p], 0, 0, dv)
                    else:
                        for g in range(2):
                            dv = head_block(e, r, rq[g], rk[g], rv[g],
                                            g * groups_cnt * C,
                                            g * groups_cnt, dv)
                    dsl = pl.ds(r * DENW, DENW)
                    accd[dsl] = accd[dsl] + dv
                return 0
            lax.fori_loop(0, CH, edge, 0)

        if ng == 1:
            @pl.when(n_ch > 0)
            def _():
                issue(0, 0, hg0)

            def pair(ip, _):
                i0 = ip * 2

                @pl.when(i0 + 1 < n_ch)
                def _():
                    issue(1, i0 + 1, hg0)
                wait(0)
                compute(0, i0)

                @pl.when(i0 + 2 < n_ch)
                def _():
                    issue(0, i0 + 2, hg0)

                @pl.when(i0 + 1 < n_ch)
                def _():
                    wait(1)
                    compute(1, i0 + 1)
                return 0
            lax.fori_loop(0, (n_ch + 1) // 2, pair, 0)
        else:
            def chunk(i, _):
                issue2(i)
                wait(0)
                wait(1)
                compute(0, i)
                return 0
            lax.fori_loop(0, n_ch, chunk, 0)

        pltpu.sync_copy(accn, num_hbm.at[pl.ds(base_node, RPW)])
        pltpu.sync_copy(accd, den_hbm.at[pl.ds(base_node * DENW, RPW * DENW)])

    scratch = [pltpu.VMEM((CH,), jnp.int32) for _ in range(10)]
    scratch += [pltpu.VMEM((CH, ROWW), jnp.float32) for _ in range(6)]
    scratch += [
        pltpu.VMEM((RPW, NUMW), jnp.float32),
        pltpu.VMEM((RPW * DENW,), jnp.float32),
        pltpu.VMEM((48,), jnp.int32),
        pltpu.SemaphoreType.DMA,
        pltpu.SemaphoreType.DMA,
    ]
    return pl.kernel(
        body,
        out_type=(jax.ShapeDtypeStruct((NPAD, NUMW), jnp.float32),
                  jax.ShapeDtypeStruct((NPAD * DENW,), jnp.float32)),
        mesh=mesh,
        compiler_params=pltpu.CompilerParams(needs_layout_passes=False),
        scratch_types=scratch,
    )


def _ln(t, g, b):
    mu = jnp.mean(t, axis=-1, keepdims=True)
    var = jnp.mean((t - mu) ** 2, axis=-1, keepdims=True)
    return (t - mu) * lax.rsqrt(var + 1e-5) * g + b


def _elu(t):
    return jnp.where(t > 0, t, jnp.exp(t) - 1.0)


def _assemble(num_refs, den_refs, s_ref, foh, C, head_cnts):
    rows = s_ref.shape[1]
    segs = []
    for num_ref, den_ref, nheads in zip(num_refs, den_refs, head_cnts):
        numw = nheads * C
        num = num_ref[:, :numw]
        den = den_ref[:, :nheads]
        den = jnp.broadcast_to(
            den[:, :, None], (rows, nheads, C)).reshape(rows, numw)
        segs.append(num / (den + 1e-16))
    t = jnp.concatenate(segs, axis=-1) if len(segs) > 1 else segs[0]
    skip = jnp.concatenate([s_ref[0][:, :foh], s_ref[1][:, :foh]], axis=-1)
    return t + skip


def _epilogue(nums, head_cnts, proj, g, b, fout, res=None):
    foh = fout // 2
    C = fout // HEADS
    ncalls = len(nums)
    grid = (N_NODES // ROW_BLOCK,)
    in_specs = []
    args = []
    for num2, den2 in nums:
        in_specs.append(pl.BlockSpec((ROW_BLOCK, NUMW), lambda i: (i, 0)))
        in_specs.append(pl.BlockSpec((ROW_BLOCK, DENW), lambda i: (i, 0)))
        args += [num2, den2]
    in_specs += [
        pl.BlockSpec((SC_CORES, ROW_BLOCK, ROWW), lambda i: (3, i, 0)),
        pl.BlockSpec((fout,), lambda i: (0,)),
        pl.BlockSpec((fout,), lambda i: (0,)),
    ]
    args += [proj, g, b]
    has_res = res is not None
    if has_res:
        in_specs.append(pl.BlockSpec((ROW_BLOCK, fout), lambda i: (i, 0)))
        args.append(res)

    def body(*refs):
        num_refs = [refs[2 * j] for j in range(ncalls)]
        den_refs = [refs[2 * j + 1] for j in range(ncalls)]
        s_ref, g_ref, b_ref = refs[2 * ncalls:2 * ncalls + 3]
        o_ref = refs[-1]
        t = _assemble(num_refs, den_refs, s_ref, foh, C, head_cnts)
        y = _elu(_ln(t, g_ref[...], b_ref[...]))
        if has_res:
            y = y + refs[2 * ncalls + 3][...]
        o_ref[...] = y

    return pl.pallas_call(
        body,
        grid=grid,
        in_specs=in_specs,
        out_specs=pl.BlockSpec((ROW_BLOCK, fout), lambda i: (i, 0)),
        out_shape=jax.ShapeDtypeStruct((N_NODES, fout), jnp.float32),
    )(*args)


def _final(nums, head_cnts, proj, g, b, wcls_p, bcls_p, fout):
    foh = fout // 2
    C = fout // HEADS
    ncalls = len(nums)
    grid = (N_NODES // ROW_BLOCK,)
    in_specs = []
    args = []
    for num2, den2 in nums:
        in_specs.append(pl.BlockSpec((ROW_BLOCK, NUMW), lambda i: (i, 0)))
        in_specs.append(pl.BlockSpec((ROW_BLOCK, DENW), lambda i: (i, 0)))
        args += [num2, den2]
    in_specs += [
        pl.BlockSpec((SC_CORES, ROW_BLOCK, ROWW), lambda i: (3, i, 0)),
        pl.BlockSpec((fout,), lambda i: (0,)),
        pl.BlockSpec((fout,), lambda i: (0,)),
        pl.BlockSpec((fout, 128), lambda i: (0, 0)),
        pl.BlockSpec((128,), lambda i: (0,)),
    ]
    args += [proj, g, b, wcls_p, bcls_p]

    def body(*refs):
        num_refs = [refs[2 * j] for j in range(ncalls)]
        den_refs = [refs[2 * j + 1] for j in range(ncalls)]
        s_ref, g_ref, b_ref, w_ref, bc_ref = refs[2 * ncalls:2 * ncalls + 5]
        lg_ref, pna_ref, obn_ref = refs[2 * ncalls + 5:]
        t = _assemble(num_refs, den_refs, s_ref, foh, C, head_cnts)
        obn = _ln(t, g_ref[...], b_ref[...])
        h = _elu(obn)
        pna_ref[...] = t
        obn_ref[...] = obn
        lg_ref[...] = (
            jnp.dot(h, w_ref[...], preferred_element_type=jnp.float32)
            + bc_ref[...]
        )

    return pl.pallas_call(
        body,
        grid=grid,
        in_specs=in_specs,
        out_specs=[
            pl.BlockSpec((ROW_BLOCK, 128), lambda i: (i, 0)),
            pl.BlockSpec((ROW_BLOCK, fout), lambda i: (i, 0)),
            pl.BlockSpec((ROW_BLOCK, fout), lambda i: (i, 0)),
        ],
        out_shape=[
            jax.ShapeDtypeStruct((N_NODES, 128), jnp.float32),
            jax.ShapeDtypeStruct((N_NODES, fout), jnp.float32),
            jax.ShapeDtypeStruct((N_NODES, fout), jnp.float32),
        ],
    )(*args)


def _weight_groups(cp):
    fout = cp['Wq'].shape[0]
    foh = fout // 2
    pad = ROWW - foh
    ws, bs = [], []
    for nm in ('q', 'k', 'v', 's'):
        wt = cp['W' + nm].T
        bb = cp['b' + nm]
        for c in range(2):
            ws.append(jnp.pad(wt[:, c * foh:(c + 1) * foh],
                              ((0, 0), (0, pad))))
            bs.append(jnp.pad(bb[c * foh:(c + 1) * foh], (0, pad)))
    return jnp.stack(ws), jnp.stack(bs)


def kernel(x, edge_index, params):
    src = edge_index[0].astype(jnp.int32)
    dst = edge_index[1].astype(jnp.int32)

    order = jnp.argsort(dst)
    sdst = dst[order]
    ssrc = src[order]
    bounds = jnp.arange(0, NPAD + 1, RPW)
    starts = jnp.searchsorted(sdst, bounds).astype(jnp.int32)
    starts = jnp.pad(starts, (0, 48 - starts.shape[0]))
    sdst_p = jnp.pad(sdst, (0, EPAD))
    ssrc_p = jnp.pad(ssrc, (0, EPAD))

    def layer(h, cp, splits):
        fout = cp['Wq'].shape[0]
        C = fout // HEADS
        wg, bg = _weight_groups(cp)
        proj = _project(h, wg, bg)
        qkvf = proj.reshape(8 * N_NODES, ROWW)
        nums, cnts = [], []
        for groups in splits:
            num, den = _edge_pass(C, groups)(qkvf, ssrc_p, sdst_p, starts)
            nums.append((num, den.reshape(NPAD, DENW)))
            cnts.append(sum(cnt for _, cnt in groups))
        return nums, tuple(cnts), proj

    full8 = (((0, 4), (1, 4)),)
    halves = (((0, 4),), ((1, 4),))
    p = params
    nums, cnts, proj = layer(x, p['conv1'], full8)
    h1 = _epilogue(nums, cnts, proj, p['ln1']['g'], p['ln1']['b'], 128)
    nums, cnts, proj = layer(h1, p['conv2'], (((0, 4),), ((1, 4),)))
    h2 = _epilogue(nums, cnts, proj, p['ln2']['g'], p['ln2']['b'], 256)
    nums, cnts, proj = layer(h2, p['conv3'], full8)
    h4_in = _epilogue(nums, cnts, proj, p['ln3']['g'], p['ln3']['b'], 128,
                      res=h1)
    nums, cnts, proj = layer(h4_in, p['conv4'], full8)
    wcls_p = jnp.zeros((64, 128), jnp.float32).at[:, :NCLS].set(
        p['cls']['W'].T)
    bcls_p = jnp.zeros((128,), jnp.float32).at[:NCLS].set(p['cls']['b'])
    logits_p, pna, obn = _final(nums, cnts, proj, p['ln4']['g'],
                                p['ln4']['b'], wcls_p, bcls_p, 64)
    return (logits_p[:, :NCLS], pna, obn)


# speedup vs baseline: 1.0636x; 1.0636x over previous
import functools
import math

import jax
import jax.numpy as jnp
from jax import lax
from jax.experimental import pallas as pl
from jax.experimental.pallas import tpu as pltpu
from jax.experimental.pallas import tpu_sc as plsc

N_NODES = 10000
N_EDGES = 320000
HEADS = 8
NCLS = 10

SC_CORES = 2
SC_TILES = 16
NW = SC_CORES * SC_TILES
LANES = 16
CH = 80
NPAD = 10240
RPW = NPAD // NW
ROW_BLOCK = 1000
ROWW = 128
NUMW = 128
DENW = 16
VW = NUMW + DENW
EPAD = 128


def _proj_body(h_ref, w_ref, b_ref, o_ref):
    o_ref[0] = (
        jnp.dot(h_ref[...], w_ref[0], preferred_element_type=jnp.float32)
        + b_ref[0]
    )


def _project(h, wg, bg):
    g_cnt, fin, roww = wg.shape
    bg = bg.reshape(g_cnt, 1, roww)
    grid = (N_NODES // ROW_BLOCK, g_cnt)
    return pl.pallas_call(
        _proj_body,
        grid=grid,
        in_specs=[
            pl.BlockSpec((ROW_BLOCK, fin), lambda i, g: (i, 0)),
            pl.BlockSpec((1, fin, roww), lambda i, g: (g, 0, 0)),
            pl.BlockSpec((1, 1, roww), lambda i, g: (g, 0, 0)),
        ],
        out_specs=pl.BlockSpec((1, ROW_BLOCK, roww), lambda i, g: (g, i, 0)),
        out_shape=jax.ShapeDtypeStruct((g_cnt, N_NODES, roww), jnp.float32),
    )(h, wg, bg)


@functools.lru_cache(maxsize=None)
def _edge_pass(C, groups):
    inv = 1.0 / math.sqrt(C)
    ((hg, cnt),) = groups
    numw = cnt * C
    assert numw <= NUMW and cnt <= DENW
    n_acc_slices = -(-numw // LANES)
    mesh = plsc.VectorSubcoreMesh(core_axis_name="c", subcore_axis_name="s",
                                  num_cores=SC_CORES, num_subcores=SC_TILES)

    def body(qkv_hbm, ssrc_hbm, sdst_hbm, starts_hbm, num_hbm, den_hbm, *scr):
        sidx = scr[0:2]
        didx = scr[2:4]
        gq = scr[4:6]
        gk = scr[6:8]
        gv = scr[8:10]
        rq = scr[10:12]
        rk = scr[12:14]
        rv = scr[14:16]
        accn, accd, starts_v = scr[16:19]
        sems = scr[19:21]

        c = lax.axis_index("c")
        s = lax.axis_index("s")
        w = c * SC_TILES + s
        zeros = jnp.zeros((LANES,), jnp.float32)
        lane = lax.iota(jnp.int32, LANES)

        pltpu.sync_copy(starts_hbm, starts_v)

        def start_at(i):
            v = jnp.zeros((), jnp.int32)
            for j in range(3):
                sl = starts_v[pl.ds(j * LANES, LANES)]
                v = v + jnp.sum(jnp.where(lane + j * LANES == i, sl, 0))
            return v

        e0 = start_at(w)
        e1 = start_at(w + 1)
        e0a = (e0 // 8) * 8
        n_ch = (e1 - e0a + CH - 1) // CH
        base_node = w * RPW

        def zacc(i, _):
            def zc(j, _):
                accn[i, pl.ds(j * LANES, LANES)] = zeros
                return 0
            lax.fori_loop(0, NUMW // LANES, zc, 0)
            accd[pl.ds(i * DENW, DENW)] = zeros
            return 0
        lax.fori_loop(0, RPW, zacc, 0)

        def issue(p, i):
            eb = e0a + i * CH
            pltpu.sync_copy(ssrc_hbm.at[pl.ds(eb, CH)], sidx[p])
            pltpu.sync_copy(sdst_hbm.at[pl.ds(eb, CH)], didx[p])
            for g4 in range(CH // LANES):
                sl = pl.ds(g4 * LANES, LANES)
                sv = sidx[p][sl]
                dv = didx[p][sl]
                gq[p][sl] = dv + hg * N_NODES
                gk[p][sl] = sv + (2 + hg) * N_NODES
                gv[p][sl] = sv + (4 + hg) * N_NODES
            pltpu.async_copy(qkv_hbm.at[gq[p]], rq[p], sems[p])
            pltpu.async_copy(qkv_hbm.at[gk[p]], rk[p], sems[p])
            pltpu.async_copy(qkv_hbm.at[gv[p]], rv[p], sems[p])

        def wait(p):
            pltpu.make_async_copy(qkv_hbm.at[gq[p]], rq[p], sems[p]).wait()
            pltpu.make_async_copy(qkv_hbm.at[gk[p]], rk[p], sems[p]).wait()
            pltpu.make_async_copy(qkv_hbm.at[gv[p]], rv[p], sems[p]).wait()

        def compute(p, i):
            eb = e0a + i * CH
            qrows, krows, vrows = rq[p], rk[p], rv[p]
            dxp = didx[p]

            def edge(e, _):
                ge = eb + e

                @pl.when(jnp.logical_and(ge >= e0, ge < e1))
                def _():
                    dsl16 = dxp[pl.ds((e // LANES) * LANES, LANES)]
                    dl = jnp.sum(jnp.where(lane == e % LANES, dsl16, 0))
                    r = dl - base_node
                    dv = jnp.zeros((LANES,), jnp.float32)
                    if C == 16:
                        for th in range(cnt):
                            sl = pl.ds(th * 16, LANES)
                            d = qrows[e, sl] * krows[e, sl]
                            a = jnp.sum(d) * inv
                            exv = jnp.exp(jnp.broadcast_to(a, (LANES,)))
                            accn[r, sl] = accn[r, sl] + vrows[e, sl] * exv
                            dv = dv + jnp.where(lane == th, exv, 0.0)
                    elif C == 32:
                        for th in range(cnt):
                            s0 = pl.ds(th * 32, LANES)
                            s1 = pl.ds(th * 32 + 16, LANES)
                            d = (qrows[e, s0] * krows[e, s0]
                                 + qrows[e, s1] * krows[e, s1])
                            a = jnp.sum(d) * inv
                            exv = jnp.exp(jnp.broadcast_to(a, (LANES,)))
                            accn[r, s0] = accn[r, s0] + vrows[e, s0] * exv
                            accn[r, s1] = accn[r, s1] + vrows[e, s1] * exv
                            dv = dv + jnp.where(lane == th, exv, 0.0)
                    else:
                        for tp in range(cnt // 2):
                            sl = pl.ds(tp * 16, LANES)
                            d = qrows[e, sl] * krows[e, sl]
                            a0 = jnp.sum(jnp.where(lane < 8, d, 0.0)) * inv
                            a1 = jnp.sum(jnp.where(lane >= 8, d, 0.0)) * inv
                            av = jnp.where(lane < 8,
                                           jnp.broadcast_to(a0, (LANES,)),
                                           jnp.broadcast_to(a1, (LANES,)))
                            exv = jnp.exp(av)
                            accn[r, sl] = accn[r, sl] + vrows[e, sl] * exv
                            x0 = jnp.sum(jnp.where(lane == 0, exv, 0.0))
                            x1 = jnp.sum(jnp.where(lane == 8, exv, 0.0))
                            dv = (dv + jnp.where(lane == 2 * tp, x0, 0.0)
                                  + jnp.where(lane == 2 * tp + 1, x1, 0.0))
                    dsl = pl.ds(r * DENW, DENW)
                    accd[dsl] = accd[dsl] + dv
                return 0
            lax.fori_loop(0, CH, edge, 0)

        @pl.when(n_ch > 0)
        def _():
            issue(0, 0)

        def pair(ip, _):
            i0 = ip * 2

            @pl.when(i0 + 1 < n_ch)
            def _():
                issue(1, i0 + 1)
            wait(0)
            compute(0, i0)

            @pl.when(i0 + 2 < n_ch)
            def _():
                issue(0, i0 + 2)

            @pl.when(i0 + 1 < n_ch)
            def _():
                wait(1)
                compute(1, i0 + 1)
            return 0
        lax.fori_loop(0, (n_ch + 1) // 2, pair, 0)

        pltpu.sync_copy(accn, num_hbm.at[pl.ds(base_node, RPW)])
        pltpu.sync_copy(accd, den_hbm.at[pl.ds(base_node * DENW, RPW * DENW)])

    scratch = [pltpu.VMEM((CH,), jnp.int32) for _ in range(10)]
    scratch += [pltpu.VMEM((CH, ROWW), jnp.float32) for _ in range(6)]
    scratch += [
        pltpu.VMEM((RPW, NUMW), jnp.float32),
        pltpu.VMEM((RPW * DENW,), jnp.float32),
        pltpu.VMEM((48,), jnp.int32),
        pltpu.SemaphoreType.DMA,
        pltpu.SemaphoreType.DMA,
    ]
    return pl.kernel(
        body,
        out_type=(jax.ShapeDtypeStruct((NPAD, NUMW), jnp.float32),
                  jax.ShapeDtypeStruct((NPAD * DENW,), jnp.float32)),
        mesh=mesh,
        compiler_params=pltpu.CompilerParams(needs_layout_passes=False),
        scratch_types=scratch,
    )


def _ln(t, g, b):
    mu = jnp.mean(t, axis=-1, keepdims=True)
    var = jnp.mean((t - mu) ** 2, axis=-1, keepdims=True)
    return (t - mu) * lax.rsqrt(var + 1e-5) * g + b


def _elu(t):
    return jnp.where(t > 0, t, jnp.exp(t) - 1.0)


def _assemble(num_refs, den_refs, s_ref, foh, C, head_cnts):
    rows = s_ref.shape[1]
    segs = []
    for num_ref, den_ref, nheads in zip(num_refs, den_refs, head_cnts):
        numw = nheads * C
        num = num_ref[:, :numw]
        den = den_ref[:, :nheads]
        den = jnp.broadcast_to(
            den[:, :, None], (rows, nheads, C)).reshape(rows, numw)
        segs.append(num / (den + 1e-16))
    t = jnp.concatenate(segs, axis=-1) if len(segs) > 1 else segs[0]
    skip = jnp.concatenate([s_ref[0][:, :foh], s_ref[1][:, :foh]], axis=-1)
    return t + skip


def _epilogue(nums, head_cnts, proj, g, b, fout, res=None):
    foh = fout // 2
    C = fout // HEADS
    ncalls = len(nums)
    grid = (N_NODES // ROW_BLOCK,)
    in_specs = []
    args = []
    for num2, den2 in nums:
        in_specs.append(pl.BlockSpec((ROW_BLOCK, NUMW), lambda i: (i, 0)))
        in_specs.append(pl.BlockSpec((ROW_BLOCK, DENW), lambda i: (i, 0)))
        args += [num2, den2]
    in_specs += [
        pl.BlockSpec((SC_CORES, ROW_BLOCK, ROWW), lambda i: (3, i, 0)),
        pl.BlockSpec((fout,), lambda i: (0,)),
        pl.BlockSpec((fout,), lambda i: (0,)),
    ]
    args += [proj, g, b]
    has_res = res is not None
    if has_res:
        in_specs.append(pl.BlockSpec((ROW_BLOCK, fout), lambda i: (i, 0)))
        args.append(res)

    def body(*refs):
        num_refs = [refs[2 * j] for j in range(ncalls)]
        den_refs = [refs[2 * j + 1] for j in range(ncalls)]
        s_ref, g_ref, b_ref = refs[2 * ncalls:2 * ncalls + 3]
        o_ref = refs[-1]
        t = _assemble(num_refs, den_refs, s_ref, foh, C, head_cnts)
        y = _elu(_ln(t, g_ref[...], b_ref[...]))
        if has_res:
            y = y + refs[2 * ncalls + 3][...]
        o_ref[...] = y

    return pl.pallas_call(
        body,
        grid=grid,
        in_specs=in_specs,
        out_specs=pl.BlockSpec((ROW_BLOCK, fout), lambda i: (i, 0)),
        out_shape=jax.ShapeDtypeStruct((N_NODES, fout), jnp.float32),
    )(*args)


def _final(nums, head_cnts, proj, g, b, wcls_p, bcls_p, fout):
    foh = fout // 2
    C = fout // HEADS
    ncalls = len(nums)
    grid = (N_NODES // ROW_BLOCK,)
    in_specs = []
    args = []
    for num2, den2 in nums:
        in_specs.append(pl.BlockSpec((ROW_BLOCK, NUMW), lambda i: (i, 0)))
        in_specs.append(pl.BlockSpec((ROW_BLOCK, DENW), lambda i: (i, 0)))
        args += [num2, den2]
    in_specs += [
        pl.BlockSpec((SC_CORES, ROW_BLOCK, ROWW), lambda i: (3, i, 0)),
        pl.BlockSpec((fout,), lambda i: (0,)),
        pl.BlockSpec((fout,), lambda i: (0,)),
        pl.BlockSpec((fout, 128), lambda i: (0, 0)),
        pl.BlockSpec((128,), lambda i: (0,)),
    ]
    args += [proj, g, b, wcls_p, bcls_p]

    def body(*refs):
        num_refs = [refs[2 * j] for j in range(ncalls)]
        den_refs = [refs[2 * j + 1] for j in range(ncalls)]
        s_ref, g_ref, b_ref, w_ref, bc_ref = refs[2 * ncalls:2 * ncalls + 5]
        lg_ref, pna_ref, obn_ref = refs[2 * ncalls + 5:]
        t = _assemble(num_refs, den_refs, s_ref, foh, C, head_cnts)
        obn = _ln(t, g_ref[...], b_ref[...])
        h = _elu(obn)
        pna_ref[...] = t
        obn_ref[...] = obn
        lg_ref[...] = (
            jnp.dot(h, w_ref[...], preferred_element_type=jnp.float32)
            + bc_ref[...]
        )

    return pl.pallas_call(
        body,
        grid=grid,
        in_specs=in_specs,
        out_specs=[
            pl.BlockSpec((ROW_BLOCK, 128), lambda i: (i, 0)),
            pl.BlockSpec((ROW_BLOCK, fout), lambda i: (i, 0)),
            pl.BlockSpec((ROW_BLOCK, fout), lambda i: (i, 0)),
        ],
        out_shape=[
            jax.ShapeDtypeStruct((N_NODES, 128), jnp.float32),
            jax.ShapeDtypeStruct((N_NODES, fout), jnp.float32),
            jax.ShapeDtypeStruct((N_NODES, fout), jnp.float32),
        ],
    )(*args)


def _weight_groups(cp):
    fout = cp['Wq'].shape[0]
    foh = fout // 2
    pad = ROWW - foh
    ws, bs = [], []
    for nm in ('q', 'k', 'v', 's'):
        wt = cp['W' + nm].T
        bb = cp['b' + nm]
        for c in range(2):
            ws.append(jnp.pad(wt[:, c * foh:(c + 1) * foh],
                              ((0, 0), (0, pad))))
            bs.append(jnp.pad(bb[c * foh:(c + 1) * foh], (0, pad)))
    return jnp.stack(ws), jnp.stack(bs)


def kernel(x, edge_index, params):
    src = edge_index[0].astype(jnp.int32)
    dst = edge_index[1].astype(jnp.int32)

    order = jnp.argsort(dst)
    sdst = dst[order]
    ssrc = src[order]
    bounds = jnp.arange(0, NPAD + 1, RPW)
    starts = jnp.searchsorted(sdst, bounds).astype(jnp.int32)
    starts = jnp.pad(starts, (0, 48 - starts.shape[0]))
    sdst_p = jnp.pad(sdst, (0, EPAD))
    ssrc_p = jnp.pad(ssrc, (0, EPAD))

    def layer(h, cp, splits):
        fout = cp['Wq'].shape[0]
        C = fout // HEADS
        wg, bg = _weight_groups(cp)
        proj = _project(h, wg, bg)
        qkvf = proj.reshape(8 * N_NODES, ROWW)
        nums, cnts = [], []
        for groups in splits:
            num, den = _edge_pass(C, groups)(qkvf, ssrc_p, sdst_p, starts)
            nums.append((num, den.reshape(NPAD, DENW)))
            cnts.append(sum(cnt for _, cnt in groups))
        return nums, tuple(cnts), proj

    halves = (((0, 4),), ((1, 4),))
    p = params
    nums, cnts, proj = layer(x, p['conv1'], halves)
    h1 = _epilogue(nums, cnts, proj, p['ln1']['g'], p['ln1']['b'], 128)
    nums, cnts, proj = layer(h1, p['conv2'], (((0, 4),), ((1, 4),)))
    h2 = _epilogue(nums, cnts, proj, p['ln2']['g'], p['ln2']['b'], 256)
    nums, cnts, proj = layer(h2, p['conv3'], halves)
    h4_in = _epilogue(nums, cnts, proj, p['ln3']['g'], p['ln3']['b'], 128,
                      res=h1)
    nums, cnts, proj = layer(h4_in, p['conv4'], halves)
    wcls_p = jnp.zeros((64, 128), jnp.float32).at[:, :NCLS].set(
        p['cls']['W'].T)
    bcls_p = jnp.zeros((128,), jnp.float32).at[:NCLS].set(p['cls']['b'])
    logits_p, pna, obn = _final(nums, cnts, proj, p['ln4']['g'],
                                p['ln4']['b'], wcls_p, bcls_p, 64)
    return (logits_p[:, :NCLS], pna, obn)
